# baseline (speedup 1.0000x reference)
import dataclasses
import functools

import jax
import jax.numpy as jnp
from jax.experimental import pallas as pl
from jax.experimental.pallas import tpu as pltpu
from jax.experimental.pallas import tpu_sc as plsc

_FILL = -1000000000.0


def _fill_body(out_ref, buf_ref, sem, *, bb):
    B = out_ref.shape[0]
    buf_ref[...] = jnp.full(buf_ref.shape, _FILL, dtype=buf_ref.dtype)
    copies = [
        pltpu.make_async_copy(buf_ref, out_ref.at[pl.ds(j * bb, bb)], sem)
        for j in range(B // bb)
    ]
    for c in copies:
        c.start()
    for c in copies:
        c.wait()


def _tc_fill(B, N, N_, dtype):
    bb = 2
    return pl.pallas_call(
        functools.partial(_fill_body, bb=bb),
        grid=(1,),
        in_specs=[],
        out_specs=pl.BlockSpec(memory_space=pltpu.MemorySpace.HBM),
        out_shape=jax.ShapeDtypeStruct((B, N, N_), dtype),
        scratch_shapes=[
            pltpu.MemorySpace.VMEM((bb, N, N_), jnp.float32),
            pltpu.SemaphoreType.DMA,
        ],
    )()


def _newton_sqrt(x):
    i = jax.lax.bitcast_convert_type(x, jnp.int32)
    magic = jnp.full(x.shape, 0x5F3759DF, jnp.int32)
    y = jax.lax.bitcast_convert_type(
        magic - jax.lax.shift_right_arithmetic(i, jnp.full(x.shape, 1, jnp.int32)),
        jnp.float32,
    )
    c15 = jnp.full(x.shape, 1.5, jnp.float32)
    ch = jnp.full(x.shape, 0.5, jnp.float32)
    for _ in range(4):
        y = y * (c15 - ch * x * y * y)
    return x * y


def _sc_sim_body(lhs_hbm, rhs_hbm, sim_hbm, lhs_vm, rhs_vm, sim_vm, sem,
                 *, N, dim, tail_start):
    core = jax.lax.axis_index("core")
    sub = jax.lax.axis_index("subcore")
    b = core * 16 + sub
    pltpu.async_copy(lhs_hbm.at[b], lhs_vm, sem).wait()
    pltpu.async_copy(rhs_hbm.at[b, pl.ds(tail_start, N)], rhs_vm, sem).wait()

    lanes = jax.lax.iota(jnp.int32, 16)
    lanes_hi = lanes + jnp.full((16,), 16, jnp.int32)
    acc_lo = jnp.zeros((16,), jnp.float32)
    acc_hi = jnp.zeros((16,), jnp.float32)
    for d in range(dim):
        col = jnp.full((16,), d, jnp.int32)
        dlo = plsc.load_gather(lhs_vm, [lanes, col]) - plsc.load_gather(
            rhs_vm, [lanes, col])
        acc_lo = acc_lo + dlo * dlo
        dhi = plsc.load_gather(lhs_vm, [lanes_hi, col]) - plsc.load_gather(
            rhs_vm, [lanes_hi, col])
        acc_hi = acc_hi + dhi * dhi

    sim_vm[pl.ds(0, 16)] = _newton_sqrt(acc_lo)
    sim_vm[pl.ds(16, 16)] = _newton_sqrt(acc_hi)
    pltpu.async_copy(sim_vm, sim_hbm.at[b], sem).wait()


def _sc_sim(lhs, rhs):
    B, N, dim = lhs.shape
    N_ = rhs.shape[1]
    mesh = plsc.VectorSubcoreMesh(core_axis_name="core",
                                  subcore_axis_name="subcore",
                                  num_cores=2, num_subcores=16)
    body = functools.partial(
        _sc_sim_body, N=N, dim=dim, tail_start=N_ - N
    )
    cp = pltpu.CompilerParams()
    if "needs_layout_passes" in pltpu.CompilerParams.__dataclass_fields__:
        cp = dataclasses.replace(cp, needs_layout_passes=False)
    kern = pl.kernel(
        body,
        out_type=jax.ShapeDtypeStruct((B, N), jnp.float32),
        mesh=mesh,
        compiler_params=cp,
        scratch_types=[
            pltpu.MemorySpace.VMEM((N, dim), jnp.float32),
            pltpu.MemorySpace.VMEM((N, dim), jnp.float32),
            pltpu.MemorySpace.VMEM((N,), jnp.float32),
            pltpu.SemaphoreType.DMA,
        ],
    )
    return kern(lhs, rhs)


def _merge_body(filled_ref, sim_ref, out_ref, tail_vm, sem, *, tail):
    del filled_ref
    B, N, N_ = out_ref.shape
    row = jax.lax.broadcasted_iota(jnp.int32, (B, N, tail), 1)
    col = jax.lax.broadcasted_iota(jnp.int32, (B, N, tail), 2)
    mask = col == row + (tail - N)
    tail_vm[...] = jnp.where(mask, sim_ref[...][:, :, None],
                             jnp.float32(_FILL))
    copy = pltpu.make_async_copy(
        tail_vm, out_ref.at[:, :, pl.ds(N_ - tail, tail)], sem
    )
    copy.start()
    copy.wait()


def _tc_merge(filled, sim, tail=128):
    B, N, N_ = filled.shape
    return pl.pallas_call(
        functools.partial(_merge_body, tail=tail),
        grid=(1,),
        in_specs=[
            pl.BlockSpec(memory_space=pltpu.MemorySpace.HBM),
            pl.BlockSpec((B, N), lambda i: (0, 0)),
        ],
        out_specs=pl.BlockSpec(memory_space=pltpu.MemorySpace.HBM),
        out_shape=jax.ShapeDtypeStruct((B, N, N_), filled.dtype),
        input_output_aliases={0: 0},
        scratch_shapes=[
            pltpu.MemorySpace.VMEM((B, N, tail), jnp.float32),
            pltpu.SemaphoreType.DMA,
        ],
    )(filled, sim)


def kernel(lhs, rhs):
    B, N, dim = lhs.shape
    N_ = rhs.shape[1]
    filled = _tc_fill(B, N, N_, lhs.dtype)
    sim = _sc_sim(lhs, rhs)
    return _tc_merge(filled, sim)


# speedup vs baseline: 1.1405x; 1.1405x over previous
import dataclasses
import functools

import jax
import jax.numpy as jnp
from jax.experimental import pallas as pl
from jax.experimental.pallas import tpu as pltpu
from jax.experimental.pallas import tpu_sc as plsc

_FILL = -1000000000.0


def _fill_body(out_ref, buf_ref, sem, *, bb):
    B = out_ref.shape[0]
    buf_ref[...] = jnp.full(buf_ref.shape, _FILL, dtype=buf_ref.dtype)
    copies = [
        pltpu.make_async_copy(buf_ref, out_ref.at[pl.ds(j * bb, bb)], sem)
        for j in range(B // bb)
    ]
    for c in copies:
        c.start()
    for c in copies:
        c.wait()


def _tc_fill(B, N, N_, dtype):
    bb = 2
    return pl.pallas_call(
        functools.partial(_fill_body, bb=bb),
        grid=(1,),
        in_specs=[],
        out_specs=pl.BlockSpec(memory_space=pltpu.MemorySpace.HBM),
        out_shape=jax.ShapeDtypeStruct((B, N, N_), dtype),
        scratch_shapes=[
            pltpu.MemorySpace.VMEM((bb, N, N_), jnp.float32),
            pltpu.SemaphoreType.DMA,
        ],
    )()


def _newton_sqrt(x):
    i = jax.lax.bitcast_convert_type(x, jnp.int32)
    magic = jnp.full(x.shape, 0x5F3759DF, jnp.int32)
    y = jax.lax.bitcast_convert_type(
        magic - jax.lax.shift_right_arithmetic(i, jnp.full(x.shape, 1, jnp.int32)),
        jnp.float32,
    )
    c15 = jnp.full(x.shape, 1.5, jnp.float32)
    ch = jnp.full(x.shape, 0.5, jnp.float32)
    for _ in range(4):
        y = y * (c15 - ch * x * y * y)
    return x * y


def _sc_sim_body(lhs_hbm, rhs_hbm, sim_hbm, lhs_vm, rhs_vm, sim_vm, sem,
                 *, N, dim, tail_start):
    core = jax.lax.axis_index("core")
    sub = jax.lax.axis_index("subcore")
    b = core * 16 + sub
    c_l = pltpu.make_async_copy(lhs_hbm.at[b], lhs_vm, sem)
    c_r = pltpu.make_async_copy(rhs_hbm.at[b, pl.ds(tail_start, N)], rhs_vm,
                                sem)
    c_l.start()
    c_r.start()
    c_l.wait()
    c_r.wait()

    lanes = jax.lax.iota(jnp.int32, 16)
    s_lo = jnp.zeros((16,), jnp.float32)
    s_hi = jnp.zeros((16,), jnp.float32)
    for i in range(N):
        acc = jnp.zeros((16,), jnp.float32)
        for c in range(dim // 16):
            d = lhs_vm[i, pl.ds(c * 16, 16)] - rhs_vm[i, pl.ds(c * 16, 16)]
            acc = acc + d * d
        sv = jnp.broadcast_to(jnp.sum(acc), (16,))
        mask = lanes == jnp.full((16,), i % 16, jnp.int32)
        if i < 16:
            s_lo = jnp.where(mask, sv, s_lo)
        else:
            s_hi = jnp.where(mask, sv, s_hi)

    sim_vm[pl.ds(0, 16)] = _newton_sqrt(s_lo)
    sim_vm[pl.ds(16, 16)] = _newton_sqrt(s_hi)
    pltpu.async_copy(sim_vm, sim_hbm.at[b], sem).wait()


def _sc_sim(lhs, rhs):
    B, N, dim = lhs.shape
    N_ = rhs.shape[1]
    mesh = plsc.VectorSubcoreMesh(core_axis_name="core",
                                  subcore_axis_name="subcore",
                                  num_cores=2, num_subcores=16)
    body = functools.partial(
        _sc_sim_body, N=N, dim=dim, tail_start=N_ - N
    )
    cp = pltpu.CompilerParams()
    if "needs_layout_passes" in pltpu.CompilerParams.__dataclass_fields__:
        cp = dataclasses.replace(cp, needs_layout_passes=False)
    kern = pl.kernel(
        body,
        out_type=jax.ShapeDtypeStruct((B, N), jnp.float32),
        mesh=mesh,
        compiler_params=cp,
        scratch_types=[
            pltpu.MemorySpace.VMEM((N, dim), jnp.float32),
            pltpu.MemorySpace.VMEM((N, dim), jnp.float32),
            pltpu.MemorySpace.VMEM((N,), jnp.float32),
            pltpu.SemaphoreType.DMA,
        ],
    )
    return kern(lhs, rhs)


def _merge_body(filled_ref, sim_ref, out_ref, tail_vm, sem, *, tail):
    del filled_ref
    B, N, N_ = out_ref.shape
    row = jax.lax.broadcasted_iota(jnp.int32, (B, N, tail), 1)
    col = jax.lax.broadcasted_iota(jnp.int32, (B, N, tail), 2)
    mask = col == row + (tail - N)
    tail_vm[...] = jnp.where(mask, sim_ref[...][:, :, None],
                             jnp.float32(_FILL))
    copy = pltpu.make_async_copy(
        tail_vm, out_ref.at[:, :, pl.ds(N_ - tail, tail)], sem
    )
    copy.start()
    copy.wait()


def _tc_merge(filled, sim, tail=128):
    B, N, N_ = filled.shape
    return pl.pallas_call(
        functools.partial(_merge_body, tail=tail),
        grid=(1,),
        in_specs=[
            pl.BlockSpec(memory_space=pltpu.MemorySpace.HBM),
            pl.BlockSpec((B, N), lambda i: (0, 0)),
        ],
        out_specs=pl.BlockSpec(memory_space=pltpu.MemorySpace.HBM),
        out_shape=jax.ShapeDtypeStruct((B, N, N_), filled.dtype),
        input_output_aliases={0: 0},
        scratch_shapes=[
            pltpu.MemorySpace.VMEM((B, N, tail), jnp.float32),
            pltpu.SemaphoreType.DMA,
        ],
    )(filled, sim)


def kernel(lhs, rhs):
    B, N, dim = lhs.shape
    N_ = rhs.shape[1]
    sim = _sc_sim(lhs, rhs)
    filled = _tc_fill(B, N, N_, lhs.dtype)
    return _tc_merge(filled, sim)


# speedup vs baseline: 2.6000x; 2.2798x over previous
import functools

import jax
import jax.numpy as jnp
from jax.experimental import pallas as pl


def _l2_band_kernel(lhs_ref, rhs_ref, out_ref, *, tail):
    bb, N, N_ = out_ref.shape
    out_ref[...] = jnp.full(out_ref.shape, -1000000000.0, dtype=out_ref.dtype)
    diff = lhs_ref[...] - rhs_ref[...]
    sim = jnp.sqrt(jnp.sum(diff * diff, axis=-1))
    row = jax.lax.broadcasted_iota(jnp.int32, (bb, N, tail), 1)
    col = jax.lax.broadcasted_iota(jnp.int32, (bb, N, tail), 2)
    mask = col == row + (tail - N)
    out_ref[:, :, N_ - tail:] = jnp.where(
        mask, sim[:, :, None], jnp.float32(-1000000000.0)
    )


def kernel(lhs, rhs):
    B, N, dim = lhs.shape
    N_ = rhs.shape[1]
    bb = 4
    tail = 128
    tail_block_idx = N_ // N - 1

    body = functools.partial(_l2_band_kernel, tail=tail)
    return pl.pallas_call(
        body,
        grid=(B // bb,),
        in_specs=[
            pl.BlockSpec((bb, N, dim), lambda j: (j, 0, 0)),
            pl.BlockSpec((bb, N, dim), lambda j: (j, tail_block_idx, 0)),
        ],
        out_specs=pl.BlockSpec((bb, N, N_), lambda j: (j, 0, 0)),
        out_shape=jax.ShapeDtypeStruct((B, N, N_), lhs.dtype),
    )(lhs, rhs)
